# baseline (speedup 1.0000x reference)
import functools

import jax
import jax.numpy as jnp
from jax import lax
from jax.experimental import pallas as pl
from jax.experimental.pallas import tpu as pltpu
from jax.experimental.pallas import tpu_sc as plsc

VOCAB = 1000000
BATCH = 4096
SEQ_LEN = 50
VEC_SIZE = 64

NUM_WORKERS = 32
CHUNK = 128


@functools.partial(jax.jit, static_argnames=())
def _gather_rows(idx_flat, table):
    n = idx_flat.shape[0]
    per_w = n // NUM_WORKERS
    n_chunks = per_w // CHUNK
    mesh = plsc.VectorSubcoreMesh(core_axis_name="c", subcore_axis_name="s")

    @functools.partial(
        pl.kernel,
        out_type=jax.ShapeDtypeStruct((n, VEC_SIZE), jnp.float32),
        mesh=mesh,
        scratch_types=[
            pltpu.VMEM((CHUNK,), jnp.int32),
            pltpu.VMEM((CHUNK, VEC_SIZE), jnp.float32),
            pltpu.SemaphoreType.DMA,
        ],
        compiler_params=pltpu.CompilerParams(use_tc_tiling_on_sc=False),
    )
    def k(idx_hbm, table_hbm, out_hbm, idx_v, rows_v, sem):
        wid = lax.axis_index("s") * 2 + lax.axis_index("c")
        base_w = wid * per_w

        @pl.loop(0, n_chunks)
        def _chunk_loop(i):
            base = base_w + i * CHUNK
            pltpu.sync_copy(idx_hbm.at[pl.ds(base, CHUNK)], idx_v)
            pltpu.async_copy(table_hbm.at[idx_v], rows_v, sem).wait()
            pltpu.sync_copy(rows_v, out_hbm.at[pl.ds(base, CHUNK)])

    return k(idx_flat, table)


def kernel(input_batch, table):
    idx_flat = input_batch.reshape(-1)
    out = _gather_rows(idx_flat, table)
    return out.reshape(BATCH, SEQ_LEN, VEC_SIZE)


# speedup vs baseline: 1.0767x; 1.0767x over previous
import functools

import jax
import jax.numpy as jnp
from jax import lax
from jax.experimental import pallas as pl
from jax.experimental.pallas import tpu as pltpu
from jax.experimental.pallas import tpu_sc as plsc

VOCAB = 1000000
BATCH = 4096
SEQ_LEN = 50
VEC_SIZE = 64

NUM_WORKERS = 32
CHUNK = 128
NBUF = 5


@jax.jit
def _gather_rows(idx3, table):
    nw, n_chunks, _ = idx3.shape
    per_w = n_chunks * CHUNK
    n = nw * per_w
    n_groups = n_chunks // NBUF
    mesh = plsc.VectorSubcoreMesh(core_axis_name="c", subcore_axis_name="s")

    @functools.partial(
        pl.kernel,
        out_type=jax.ShapeDtypeStruct((n, VEC_SIZE), jnp.float32),
        mesh=mesh,
        scratch_types=(
            [pltpu.VMEM((n_chunks, CHUNK), jnp.int32)]
            + [pltpu.VMEM((CHUNK, VEC_SIZE), jnp.float32)] * NBUF
            + [pltpu.SemaphoreType.DMA] * (2 * NBUF)
        ),
        compiler_params=pltpu.CompilerParams(use_tc_tiling_on_sc=False),
    )
    def k(idx_hbm, table_hbm, out_hbm, idx_v, *rest):
        rows = rest[:NBUF]
        sem_g = rest[NBUF : 2 * NBUF]
        sem_s = rest[2 * NBUF :]
        wid = lax.axis_index("s") * 2 + lax.axis_index("c")
        base_w = wid * per_w

        pltpu.sync_copy(idx_hbm.at[wid], idx_v)

        def start_gather(chunk, b):
            pltpu.async_copy(table_hbm.at[idx_v.at[chunk]], rows[b], sem_g[b])

        def wait_gather(chunk, b):
            pltpu.make_async_copy(
                table_hbm.at[idx_v.at[chunk]], rows[b], sem_g[b]
            ).wait()

        def start_scatter(chunk, b):
            dst = out_hbm.at[pl.ds(base_w + chunk * CHUNK, CHUNK)]
            pltpu.async_copy(rows[b], dst, sem_s[b])

        def wait_scatter(chunk, b):
            dst = out_hbm.at[pl.ds(base_w + chunk * CHUNK, CHUNK)]
            pltpu.make_async_copy(rows[b], dst, sem_s[b]).wait()

        for b in range(NBUF):
            start_gather(b, b)

        @pl.loop(0, n_groups - 1)
        def _grp(g):
            c0 = g * NBUF
            for b in range(NBUF):
                wait_gather(c0 + b, b)
                start_scatter(c0 + b, b)
            for b in range(NBUF):
                wait_scatter(c0 + b, b)
                start_gather(c0 + NBUF + b, b)

        c0 = (n_groups - 1) * NBUF
        for b in range(NBUF):
            wait_gather(c0 + b, b)
            start_scatter(c0 + b, b)
        for b in range(NBUF):
            wait_scatter(c0 + b, b)

    return k(idx3, table)


def kernel(input_batch, table):
    idx3 = input_batch.reshape(NUM_WORKERS, -1, CHUNK)
    out = _gather_rows(idx3, table)
    return out.reshape(BATCH, SEQ_LEN, VEC_SIZE)
